# baseline (speedup 1.0000x reference)
import functools

import jax
import jax.numpy as jnp
from jax import lax
from jax.experimental import pallas as pl
from jax.experimental.pallas import tpu as pltpu
from jax.experimental.pallas import tpu_sc as plsc

_B = 1024
_NFEAT = 128
_TDIM = 32
_NCLASS = 64
_F1, _F2 = 16, 16
_BF1, _BF2 = 16, 16

_NW = 32
_RPW = _B // _NW
_IPW = _RPW * _BF1
_ICH = _IPW // 128


_RG = 2
_NFI = _RG * _TDIM * _BF1 * 16
_FIR = _NFI // 128


def _sc_body(idx_hbm, table_hbm, out_hbm, idx_v, idx2_v, fidx_v, dst_v,
             res_v, sem):
    wid = lax.axis_index("s") * 2 + lax.axis_index("c")
    pltpu.sync_copy(idx_hbm.at[wid], idx_v)
    for r in range(_RPW):
        idx2_v[r, :] = idx_v[r // 8, pl.ds((r % 8) * 16, 16)]

    lanes = lax.iota(jnp.int32, 16)
    for rg in range(_RG):
        for j in range(_BF1):
            vj = plsc.load_gather(idx2_v, [lanes + rg * 16,
                                           jnp.full((16,), j, jnp.int32)])
            for d in range(_TDIM):
                p = (((rg * _TDIM) + d) * _BF1 + j) * 16
                fidx_v[p // 128, pl.ds(p % 128, 16)] = vj + d * 1000000

    copies = [
        pltpu.async_copy(
            table_hbm.at[fidx_v.at[k]],
            dst_v.at[pl.ds(k * 128, 128)],
            sem,
        )
        for k in range(_FIR)
    ]
    for cp in copies:
        cp.wait()

    for rg in range(_RG):
        for d in range(_TDIM):
            base = ((rg * _TDIM) + d) * _BF1 * 16
            a = dst_v[pl.ds(base, 16)]
            for j in range(1, _BF1):
                a = jnp.maximum(a, dst_v[pl.ds(base + j * 16, 16)])
            res_v[rg * _TDIM + d, :] = a

    for rg in range(_RG):
        pltpu.sync_copy(
            res_v.at[pl.ds(rg * _TDIM, _TDIM)],
            out_hbm.at[wid * _RG + rg],
        )


@functools.cache
def _sc_gather_max():
    return functools.partial(
        pl.kernel,
        out_type=jax.ShapeDtypeStruct((_NW * _RG, _TDIM, 16), jnp.float32),
        mesh=plsc.VectorSubcoreMesh(core_axis_name="c", subcore_axis_name="s"),
        scratch_types=[
            pltpu.VMEM((_ICH, 128), jnp.int32),
            pltpu.VMEM((_RPW, _BF1), jnp.int32),
            pltpu.VMEM((_FIR, 128), jnp.int32),
            pltpu.VMEM((_NFI,), jnp.float32),
            pltpu.VMEM((_RG * _TDIM, 16), jnp.float32),
            pltpu.SemaphoreType.DMA,
        ],
        compiler_params=pltpu.CompilerParams(needs_layout_passes=False),
    )(_sc_body)

_RB = 32
_GRID = _B // _RB


def _tc_heavy_body(x0_ref, x1_ref, x2_ref, xbi2_ref, w_ref,
                   e0_ref, e1_ref, e2_ref, b2_ref):
    w = w_ref[...].astype(jnp.bfloat16)

    def pooled(h, groups):
        return jnp.max(h.reshape(_RB, groups, _TDIM), axis=1)

    e0_ref[...] = jnp.dot(x0_ref[...].astype(jnp.bfloat16), w,
                          preferred_element_type=jnp.float32)
    h1 = jnp.dot(x1_ref[...].astype(jnp.bfloat16), w,
                 preferred_element_type=jnp.float32)
    e1_ref[...] = pooled(h1, _F1)
    h2 = jnp.dot(x2_ref[...].astype(jnp.bfloat16), w,
                 preferred_element_type=jnp.float32)
    e2_ref[...] = pooled(h2, _F1 * _F2)
    hb2 = jnp.dot(xbi2_ref[...].astype(jnp.bfloat16), w,
                  preferred_element_type=jnp.float32)
    b2_ref[...] = pooled(hb2, _BF1 * _BF2)


def _tc_heavy(x_0, x_1, x_2, xbi_2, w):
    o = jax.ShapeDtypeStruct((_B, _TDIM), jnp.float32)
    return pl.pallas_call(
        _tc_heavy_body,
        grid=(_GRID,),
        in_specs=[
            pl.BlockSpec((_RB, _NFEAT), lambda i: (i, 0)),
            pl.BlockSpec((_RB * _F1, _NFEAT), lambda i: (i, 0)),
            pl.BlockSpec((_RB * _F1 * _F2, _NFEAT), lambda i: (i, 0)),
            pl.BlockSpec((_RB * _BF1 * _BF2, _NFEAT), lambda i: (i, 0)),
            pl.BlockSpec((_NFEAT, _TDIM), lambda i: (0, 0)),
        ],
        out_specs=[pl.BlockSpec((_RB, _TDIM), lambda i: (i, 0))] * 4,
        out_shape=[o, o, o, o],
        compiler_params=pltpu.CompilerParams(
            dimension_semantics=("arbitrary",),
        ),
    )(x_0, x_1, x_2, xbi_2, w)


def _tc_tail_body(e0_ref, e1_ref, e2_ref, b1_ref, b2_ref,
                  l1w_ref, l1b_ref, l2w_ref, l2b_ref, out_ref):
    e0, e1, e2 = e0_ref[...], e1_ref[...], e2_ref[...]
    b1, b2 = b1_ref[...], b2_ref[...]
    cross = [e0 * b1, e0 * b2, e1 * b1, e1 * b2, e2 * b1, e2 * b2, e0, e1, e2]
    l1w = l1w_ref[...]
    l1b = l1b_ref[0, 0]
    att = [jnp.sum(c * l1w, axis=1, keepdims=True) + l1b for c in cross]
    m = att[0]
    for a in att[1:]:
        m = jnp.maximum(m, a)
    ex = [jnp.exp(a - m) for a in att]
    s = ex[0]
    for e in ex[1:]:
        s = s + e
    inv = 1.0 / s
    hidden = cross[0] * (ex[0] * inv)
    for c, e in zip(cross[1:], ex[1:]):
        hidden = hidden + c * (e * inv)

    out = jnp.dot(hidden, l2w_ref[...], preferred_element_type=jnp.float32)
    out = out + l2b_ref[...]
    om = jnp.max(out, axis=1, keepdims=True)
    out = out - om
    out_ref[...] = out - jnp.log(jnp.sum(jnp.exp(out), axis=1, keepdims=True))


def _tc_tail(e0, e1, e2, b1, b2, l1w, l1b, l2w, l2b):
    full = pl.BlockSpec((_B, _TDIM), lambda: (0, 0))
    return pl.pallas_call(
        _tc_tail_body,
        in_specs=[
            full, full, full, full, full,
            pl.BlockSpec((1, _TDIM), lambda: (0, 0)),
            pl.BlockSpec((1, 1), lambda: (0, 0)),
            pl.BlockSpec((_TDIM, _NCLASS), lambda: (0, 0)),
            pl.BlockSpec((1, _NCLASS), lambda: (0, 0)),
        ],
        out_specs=pl.BlockSpec((_B, _NCLASS), lambda: (0, 0)),
        out_shape=jax.ShapeDtypeStruct((_B, _NCLASS), jnp.float32),
    )(e0, e1, e2, b1, b2, l1w, l1b, l2w, l2b)


def kernel(x_0, x_1, x_2, xbi_0, xbi_1, xbi_2, weight_trans, table,
           lin1_w, lin1_b, lin2_w, lin2_b):
    del xbi_0
    idx = xbi_1.astype(jnp.int32).reshape(_NW, _ICH, 128)
    b1r = _sc_gather_max()(idx, table.T.reshape(32000000))
    b1 = b1r.transpose(0, 2, 1).reshape(_B, _TDIM)
    e0, e1, e2, b2 = _tc_heavy(x_0, x_1, x_2, xbi_2, weight_trans)
    return _tc_tail(
        e0, e1, e2, b1, b2,
        lin1_w.reshape(1, _TDIM), lin1_b.reshape(1, 1),
        lin2_w, lin2_b.reshape(1, _NCLASS),
    )


# speedup vs baseline: 6.4304x; 6.4304x over previous
import functools

import jax
import jax.numpy as jnp
from jax import lax
from jax.experimental import pallas as pl
from jax.experimental.pallas import tpu as pltpu
from jax.experimental.pallas import tpu_sc as plsc

_B = 1024
_NFEAT = 128
_TDIM = 32
_NCLASS = 64
_F1, _F2 = 16, 16
_BF1, _BF2 = 16, 16

_NW = 32
_RPW = _B // _NW
_IPW = _RPW * _BF1
_ICH = _IPW // 128


def _sc_body(idx_hbm, table_hbm, out_hbm, idx_v, tidx_v, sub_v, tiles_v,
             res_v, sem):
    wid = lax.axis_index("s") * 2 + lax.axis_index("c")
    pltpu.sync_copy(idx_hbm.at[wid], idx_v)
    for t in range(_RPW):
        v = idx_v[t // 8, pl.ds((t % 8) * 16, 16)]
        tidx_v[t, :] = v >> 3
        sub_v[t, :] = v & 7

    def root_body(r, carry):
        tv = tidx_v[r, :]
        sv = sub_v[r, :]
        copies = [
            pltpu.async_copy(
                table_hbm.at[pl.ds(tv[j] * 8, 8)], tiles_v.at[j], sem)
            for j in range(_BF1)
        ]
        for cp in copies:
            cp.wait()
        a0 = tiles_v[0, sv[0], pl.ds(0, 16)]
        a1 = tiles_v[0, sv[0], pl.ds(16, 16)]
        for j in range(1, _BF1):
            sj = sv[j]
            a0 = jnp.maximum(a0, tiles_v[j, sj, pl.ds(0, 16)])
            a1 = jnp.maximum(a1, tiles_v[j, sj, pl.ds(16, 16)])
        res_v[r, pl.ds(0, 16)] = a0
        res_v[r, pl.ds(16, 16)] = a1
        return carry

    lax.fori_loop(0, _RPW, root_body, 0)
    pltpu.sync_copy(res_v, out_hbm.at[pl.ds(wid * _RPW, _RPW)])


@functools.cache
def _sc_gather_max():
    return functools.partial(
        pl.kernel,
        out_type=jax.ShapeDtypeStruct((_B, _TDIM), jnp.float32),
        mesh=plsc.VectorSubcoreMesh(core_axis_name="c", subcore_axis_name="s"),
        scratch_types=[
            pltpu.VMEM((_ICH, 128), jnp.int32),
            pltpu.VMEM((_RPW, _BF1), jnp.int32),
            pltpu.VMEM((_RPW, _BF1), jnp.int32),
            pltpu.VMEM((_BF1, 8, _TDIM), jnp.float32),
            pltpu.VMEM((_RPW, _TDIM), jnp.float32),
            pltpu.SemaphoreType.DMA,
        ],
    )(_sc_body)

_RB = 32
_GRID = _B // _RB


def _tc_heavy_body(x0_ref, x1_ref, x2_ref, xbi2_ref, w_ref,
                   e0_ref, e1_ref, e2_ref, b2_ref):
    w = w_ref[...].astype(jnp.bfloat16)

    def pooled(h, groups):
        return jnp.max(h.reshape(_RB, groups, _TDIM), axis=1)

    e0_ref[...] = jnp.dot(x0_ref[...].astype(jnp.bfloat16), w,
                          preferred_element_type=jnp.float32)
    h1 = jnp.dot(x1_ref[...].astype(jnp.bfloat16), w,
                 preferred_element_type=jnp.float32)
    e1_ref[...] = pooled(h1, _F1)
    h2 = jnp.dot(x2_ref[...].astype(jnp.bfloat16), w,
                 preferred_element_type=jnp.float32)
    e2_ref[...] = pooled(h2, _F1 * _F2)
    hb2 = jnp.dot(xbi2_ref[...].astype(jnp.bfloat16), w,
                  preferred_element_type=jnp.float32)
    b2_ref[...] = pooled(hb2, _BF1 * _BF2)


def _tc_heavy(x_0, x_1, x_2, xbi_2, w):
    o = jax.ShapeDtypeStruct((_B, _TDIM), jnp.float32)
    return pl.pallas_call(
        _tc_heavy_body,
        grid=(_GRID,),
        in_specs=[
            pl.BlockSpec((_RB, _NFEAT), lambda i: (i, 0)),
            pl.BlockSpec((_RB * _F1, _NFEAT), lambda i: (i, 0)),
            pl.BlockSpec((_RB * _F1 * _F2, _NFEAT), lambda i: (i, 0)),
            pl.BlockSpec((_RB * _BF1 * _BF2, _NFEAT), lambda i: (i, 0)),
            pl.BlockSpec((_NFEAT, _TDIM), lambda i: (0, 0)),
        ],
        out_specs=[pl.BlockSpec((_RB, _TDIM), lambda i: (i, 0))] * 4,
        out_shape=[o, o, o, o],
        compiler_params=pltpu.CompilerParams(
            dimension_semantics=("arbitrary",),
        ),
    )(x_0, x_1, x_2, xbi_2, w)


def _tc_tail_body(e0_ref, e1_ref, e2_ref, b1_ref, b2_ref,
                  l1w_ref, l1b_ref, l2w_ref, l2b_ref, out_ref):
    e0, e1, e2 = e0_ref[...], e1_ref[...], e2_ref[...]
    b1, b2 = b1_ref[...], b2_ref[...]
    cross = [e0 * b1, e0 * b2, e1 * b1, e1 * b2, e2 * b1, e2 * b2, e0, e1, e2]
    l1w = l1w_ref[...]
    l1b = l1b_ref[0, 0]
    att = [jnp.sum(c * l1w, axis=1, keepdims=True) + l1b for c in cross]
    m = att[0]
    for a in att[1:]:
        m = jnp.maximum(m, a)
    ex = [jnp.exp(a - m) for a in att]
    s = ex[0]
    for e in ex[1:]:
        s = s + e
    inv = 1.0 / s
    hidden = cross[0] * (ex[0] * inv)
    for c, e in zip(cross[1:], ex[1:]):
        hidden = hidden + c * (e * inv)

    out = jnp.dot(hidden, l2w_ref[...], preferred_element_type=jnp.float32)
    out = out + l2b_ref[...]
    om = jnp.max(out, axis=1, keepdims=True)
    out = out - om
    out_ref[...] = out - jnp.log(jnp.sum(jnp.exp(out), axis=1, keepdims=True))


def _tc_tail(e0, e1, e2, b1, b2, l1w, l1b, l2w, l2b):
    full = pl.BlockSpec((_B, _TDIM), lambda: (0, 0))
    return pl.pallas_call(
        _tc_tail_body,
        in_specs=[
            full, full, full, full, full,
            pl.BlockSpec((1, _TDIM), lambda: (0, 0)),
            pl.BlockSpec((1, 1), lambda: (0, 0)),
            pl.BlockSpec((_TDIM, _NCLASS), lambda: (0, 0)),
            pl.BlockSpec((1, _NCLASS), lambda: (0, 0)),
        ],
        out_specs=pl.BlockSpec((_B, _NCLASS), lambda: (0, 0)),
        out_shape=jax.ShapeDtypeStruct((_B, _NCLASS), jnp.float32),
    )(e0, e1, e2, b1, b2, l1w, l1b, l2w, l2b)


def kernel(x_0, x_1, x_2, xbi_0, xbi_1, xbi_2, weight_trans, table,
           lin1_w, lin1_b, lin2_w, lin2_b):
    del xbi_0
    idx = xbi_1.astype(jnp.int32).reshape(_NW, _ICH, 128)
    b1 = _sc_gather_max()(idx, table)
    e0, e1, e2, b2 = _tc_heavy(x_0, x_1, x_2, xbi_2, weight_trans)
    return _tc_tail(
        e0, e1, e2, b1, b2,
        lin1_w.reshape(1, _TDIM), lin1_b.reshape(1, 1),
        lin2_w, lin2_b.reshape(1, _NCLASS),
    )
